# baseline (speedup 1.0000x reference)
import jax
import jax.numpy as jnp
from jax.experimental import pallas as pl
from jax.experimental.pallas import tpu as pltpu

D = 64
ALPHA = 0.2


def _leaky(x):
    return jnp.where(x >= 0, x, ALPHA * x)


def _hgat_kernel(x_ref, adj_ref, wtb_ref, vecs_ref, pw_ref, out_ref):
    x = x_ref[:]
    adj = adj_ref[0].astype(jnp.bfloat16)
    adj_t = adj.T
    vecs = vecs_ref[:]
    e_cnt = adj.shape[0]

    def layer(xin, Wt, rr, r2, q):
        xt = xin if Wt is None else jnp.dot(xin, Wt,
                                            preferred_element_type=jnp.float32)
        n_col = jax.lax.dot_general(xin, rr[0:1], (((1,), (1,)), ((), ())),
                                    preferred_element_type=jnp.float32)
        qn_col = jax.lax.dot_general(xin, rr[1:2], (((1,), (1,)), ((), ())),
                                     preferred_element_type=jnp.float32)
        e_col = _leaky(q + n_col)
        w_col = jnp.exp(e_col - jnp.max(e_col))
        xw = jnp.concatenate([xt * w_col, w_col], axis=1)
        nd = jnp.dot(adj, xw.astype(jnp.bfloat16),
                     preferred_element_type=jnp.float32)
        den = nd[:, D:D + 1]
        mean_xt = jnp.mean(xt, axis=0, keepdims=True)
        safe = den > 0
        rden = 1.0 / jnp.where(safe, den, 1.0)
        edge = jnp.where(safe, nd[:, :D] * rden, mean_xt)
        ye_row = jax.lax.dot_general(r2, edge, (((1,), (1,)), ((), ())),
                                     preferred_element_type=jnp.float32)
        m2_col = _leaky(qn_col + jnp.max(ye_row))
        a1 = jnp.exp(qn_col - m2_col).astype(jnp.bfloat16)
        a2 = jnp.exp(ALPHA * qn_col - m2_col).astype(jnp.bfloat16)
        b1 = jnp.exp(ye_row).astype(jnp.bfloat16)
        b2 = jnp.exp(ALPHA * ye_row).astype(jnp.bfloat16)
        P = jnp.maximum(a1 * b1, a2 * b2) * adj_t
        edge_aug = jnp.concatenate([edge, jnp.ones((e_cnt, 1), jnp.float32)],
                                   axis=1).astype(jnp.bfloat16)
        ndn = jnp.dot(P, edge_aug,
                      preferred_element_type=jnp.float32)
        colsum = ndn[:, D:D + 1]
        mean_edge = jnp.mean(edge, axis=0, keepdims=True)
        safe2 = colsum > 0
        rcol = 1.0 / jnp.where(safe2, colsum, 1.0)
        return jnp.where(safe2, ndn[:, :D] * rcol, mean_edge)

    q1 = jnp.sum(vecs[6:7, 0:1])
    q2 = jnp.sum(vecs[6:7, 1:2])
    h = layer(x, None, vecs[0:2], vecs[2:3], q1)
    h = jnp.where(h > 0, h, jnp.exp(h) - 1.0)
    h2 = layer(h, wtb_ref[:], vecs[3:5], vecs[5:6], q2)
    pooled = jnp.sum(h2, axis=0, keepdims=True)
    pooled_aug = jnp.concatenate([pooled, jnp.ones((1, 1), jnp.float32),
                                  jnp.zeros((1, 7), jnp.float32)],
                                 axis=1)
    logits = jnp.dot(pooled_aug, pw_ref[:],
                     preferred_element_type=jnp.float32)
    lane = jax.lax.broadcasted_iota(jnp.int32, logits.shape, 1)
    valid = lane < 16
    lmax = jnp.max(jnp.where(valid, logits, -jnp.inf))
    lse = jnp.log(jnp.sum(jnp.where(valid, jnp.exp(logits - lmax), 0.0)))
    out_ref[0] = logits - lmax - lse


def _hgat(x, adj, l2_W, vecs, pw_aug):
    B, E, N = adj.shape[0], adj.shape[1], adj.shape[2]
    grid = (B,)
    out = pl.pallas_call(
        _hgat_kernel,
        grid=grid,
        in_specs=[
            pl.BlockSpec((N, D), lambda b: (b, 0)),
            pl.BlockSpec((1, E, N), lambda b: (b, 0, 0)),
            pl.BlockSpec((D, D), lambda b: (0, 0)),
            pl.BlockSpec((16, D), lambda b: (0, 0)),
            pl.BlockSpec((D + 8, 128), lambda b: (0, 0)),
        ],
        out_specs=pl.BlockSpec((1, 1, 128), lambda b: (b, 0, 0)),
        out_shape=jax.ShapeDtypeStruct((B, 1, 128), jnp.float32),
        compiler_params=pltpu.CompilerParams(
            dimension_semantics=("parallel",)),
    )(x, adj, l2_W, vecs, pw_aug)
    return out[:, 0, :16]


def kernel(words2ids, paris_mat, table, l1_W2, l1_W3, l1_ctx, l1_a, l1_a2,
           l2_W, l2_W2, l2_W3, l2_ctx, l2_a, l2_a2, pW, pb):
    B, N = words2ids.shape
    x = table[words2ids.reshape(-1)]
    vecs = jnp.zeros((16, D), jnp.float32)
    rows = [l1_W2 @ l1_a[D:], l1_W2 @ l1_a2[:D], l1_W3 @ l1_a2[D:],
            l2_W2 @ l2_a[D:], l2_W2 @ l2_a2[:D], l2_W3 @ l2_a2[D:]]
    vecs = vecs.at[:6].set(jnp.stack(rows))
    q1 = jnp.dot(l1_ctx, l1_a[:D])
    q2 = jnp.dot(l2_ctx, l2_a[:D])
    vecs = vecs.at[6, 0].set(q1).at[6, 1].set(q2)
    pw_aug = jnp.zeros((D + 8, 128), jnp.float32)
    pw_aug = pw_aug.at[:D, :16].set(pW).at[D, :16].set(pb)
    return _hgat(x, paris_mat, l2_W, vecs, pw_aug)


# speedup vs baseline: 1.1138x; 1.1138x over previous
import jax
import jax.numpy as jnp
from jax.experimental import pallas as pl
from jax.experimental.pallas import tpu as pltpu

D = 64
ALPHA = 0.2


def _leaky(x):
    return jnp.where(x >= 0, x, ALPHA * x)


def _hgat_kernel(x_ref, adj_ref, w2a_ref, w3a_ref, wtb_ref, w2b_ref,
                 w3b_ref, vecs_ref, pw_ref, out_ref):
    x = x_ref[:]
    adj = adj_ref[0].astype(jnp.bfloat16)
    adj_t = adj.T
    vecs = vecs_ref[:]
    e_cnt = adj.shape[0]

    def layer(xin, Wt, W2, W3, a_lo, ctx, a_hi, a2_lo, a2_hi):
        x4 = jnp.dot(xin, W2, preferred_element_type=jnp.float32)
        xt = xin if Wt is None else jnp.dot(xin, Wt,
                                            preferred_element_type=jnp.float32)
        q = jnp.sum(ctx * a_lo)
        n_col = jax.lax.dot_general(x4, a_hi, (((1,), (1,)), ((), ())),
                                    preferred_element_type=jnp.float32)
        e_col = _leaky(q + n_col)
        w_col = jnp.exp(e_col - jnp.max(e_col))
        xw = jnp.concatenate([xt * w_col, w_col], axis=1)
        nd = jnp.dot(adj, xw.astype(jnp.bfloat16),
                     preferred_element_type=jnp.float32)
        den = nd[:, D:D + 1]
        mean_xt = jnp.mean(xt, axis=0, keepdims=True)
        safe = den > 0
        rden = 1.0 / jnp.where(safe, den, 1.0)
        edge = jnp.where(safe, nd[:, :D] * rden, mean_xt)
        edge4 = jnp.dot(edge, W3, preferred_element_type=jnp.float32)
        qn_col = jax.lax.dot_general(x4, a2_lo, (((1,), (1,)), ((), ())),
                                     preferred_element_type=jnp.float32)
        ye_row = jax.lax.dot_general(a2_hi, edge4, (((1,), (1,)), ((), ())),
                                     preferred_element_type=jnp.float32)
        m2_col = _leaky(qn_col + jnp.max(ye_row))
        a1 = jnp.exp(qn_col - m2_col).astype(jnp.bfloat16)
        a2 = jnp.exp(ALPHA * qn_col - m2_col).astype(jnp.bfloat16)
        b1 = jnp.exp(ye_row).astype(jnp.bfloat16)
        b2 = jnp.exp(ALPHA * ye_row).astype(jnp.bfloat16)
        P = jnp.maximum(a1 * b1, a2 * b2) * adj_t
        edge_aug = jnp.concatenate([edge, jnp.ones((e_cnt, 1), jnp.float32)],
                                   axis=1).astype(jnp.bfloat16)
        ndn = jnp.dot(P, edge_aug,
                      preferred_element_type=jnp.float32)
        colsum = ndn[:, D:D + 1]
        mean_edge = jnp.mean(edge, axis=0, keepdims=True)
        safe2 = colsum > 0
        rcol = 1.0 / jnp.where(safe2, colsum, 1.0)
        return jnp.where(safe2, ndn[:, :D] * rcol, mean_edge)

    h = layer(x, None, w2a_ref[:], w3a_ref[:],
              vecs[0:1], vecs[1:2], vecs[2:3], vecs[3:4], vecs[4:5])
    h = jnp.where(h > 0, h, jnp.exp(h) - 1.0)
    h2 = layer(h, wtb_ref[:], w2b_ref[:], w3b_ref[:],
               vecs[5:6], vecs[6:7], vecs[7:8], vecs[8:9], vecs[9:10])
    pooled = jnp.sum(h2, axis=0, keepdims=True)
    pooled_aug = jnp.concatenate([pooled, jnp.ones((1, 1), jnp.float32),
                                  jnp.zeros((1, 7), jnp.float32)],
                                 axis=1)
    logits = jnp.dot(pooled_aug, pw_ref[:],
                     preferred_element_type=jnp.float32)
    lane = jax.lax.broadcasted_iota(jnp.int32, logits.shape, 1)
    valid = lane < 16
    lmax = jnp.max(jnp.where(valid, logits, -jnp.inf))
    lse = jnp.log(jnp.sum(jnp.where(valid, jnp.exp(logits - lmax), 0.0)))
    out_ref[0] = logits - lmax - lse


def _hgat(x, adj, l1_W2, l1_W3, l2_W, l2_W2, l2_W3, vecs, pw_aug):
    B, E, N = adj.shape[0], adj.shape[1], adj.shape[2]
    grid = (B,)
    wspec = pl.BlockSpec((D, D), lambda b: (0, 0))
    out = pl.pallas_call(
        _hgat_kernel,
        grid=grid,
        in_specs=[
            pl.BlockSpec((N, D), lambda b: (b, 0)),
            pl.BlockSpec((1, E, N), lambda b: (b, 0, 0)),
            wspec, wspec, wspec, wspec, wspec,
            pl.BlockSpec((16, D), lambda b: (0, 0)),
            pl.BlockSpec((D + 8, 128), lambda b: (0, 0)),
        ],
        out_specs=pl.BlockSpec((1, 1, 128), lambda b: (b, 0, 0)),
        out_shape=jax.ShapeDtypeStruct((B, 1, 128), jnp.float32),
        compiler_params=pltpu.CompilerParams(
            dimension_semantics=("parallel",)),
    )(x, adj, l1_W2, l1_W3, l2_W, l2_W2, l2_W3, vecs, pw_aug)
    return out[:, 0, :16]


def kernel(words2ids, paris_mat, table, l1_W2, l1_W3, l1_ctx, l1_a, l1_a2,
           l2_W, l2_W2, l2_W3, l2_ctx, l2_a, l2_a2, pW, pb):
    B, N = words2ids.shape
    x = table[words2ids.reshape(-1)]
    vecs = jnp.zeros((16, D), jnp.float32)
    rows = [l1_a[:D], l1_ctx, l1_a[D:], l1_a2[:D], l1_a2[D:],
            l2_a[:D], l2_ctx, l2_a[D:], l2_a2[:D], l2_a2[D:]]
    vecs = vecs.at[:10].set(jnp.stack(rows))
    pw_aug = jnp.zeros((D + 8, 128), jnp.float32)
    pw_aug = pw_aug.at[:D, :16].set(pW).at[D, :16].set(pb)
    return _hgat(x, paris_mat, l1_W2, l1_W3, l2_W, l2_W2, l2_W3, vecs, pw_aug)
